# baseline (speedup 1.0000x reference)
import jax
import jax.numpy as jnp
from jax import lax
from jax.experimental import pallas as pl
from jax.experimental.pallas import tpu as pltpu
from jax.experimental.pallas import tpu_sc as plsc

B, T = 4096, 200
N = B * T
LOC_DIM = 64
SMALL_DIM = 48
OUT_DIM = LOC_DIM + SMALL_DIM

NC, NS = 2, 16
NW = NC * NS
SEG = 128
SEGS_PER_CHUNK = 4
CHUNK = SEG * SEGS_PER_CHUNK
PER_W = N // NW
CHUNKS_PER_W = PER_W // CHUNK
PAIRS_PER_W = CHUNKS_PER_W // 2
ROWS_PER_W = PER_W // SEG


def _sc_kernel(src_h, week_h, hour_h, dur_h, loc_t, week_f, hour_f, dur_f, out_h,
               src_a, week_a, hour_a, dur_a, loc_rows_a, small_rows_a,
               src_b, week_b, hour_b, dur_b, loc_rows_b, small_rows_b,
               week_tv, hour_tv, dur_tv,
               isem_a, isem_b, gsem_a, gsem_b, wsem_a, wsem_b):
    wid = lax.axis_index("s") * NC + lax.axis_index("c")
    row0 = wid * ROWS_PER_W

    slot_a = (src_a, week_a, hour_a, dur_a, loc_rows_a, small_rows_a,
              isem_a, gsem_a, wsem_a)
    slot_b = (src_b, week_b, hour_b, dur_b, loc_rows_b, small_rows_b,
              isem_b, gsem_b, wsem_b)

    pltpu.sync_copy(week_f, week_tv)
    pltpu.sync_copy(hour_f, hour_tv)
    pltpu.sync_copy(dur_f, dur_tv)

    def stage_idx(ci, slot):
        src_v, week_v, hour_v, dur_v, isem = slot[0], slot[1], slot[2], slot[3], slot[6]
        r = row0 + ci * SEGS_PER_CHUNK
        pltpu.async_copy(src_h.at[pl.ds(r, SEGS_PER_CHUNK)], src_v, isem)
        pltpu.async_copy(week_h.at[pl.ds(r, SEGS_PER_CHUNK)], week_v, isem)
        pltpu.async_copy(hour_h.at[pl.ds(r, SEGS_PER_CHUNK)], hour_v, isem)
        pltpu.async_copy(dur_h.at[pl.ds(r, SEGS_PER_CHUNK)], dur_v, isem)

    def wait_idx(slot):
        isem = slot[6]
        for v in (slot[0], slot[1], slot[2], slot[3]):
            pltpu.make_async_copy(src_h.at[pl.ds(0, SEGS_PER_CHUNK)], v, isem).wait()

    def wait_writes(ci, slot):
        loc_rows, small_rows, wsem = slot[4], slot[5], slot[8]
        tok0 = (row0 + ci * SEGS_PER_CHUNK) * SEG
        rows = pl.ds(tok0, CHUNK)
        pltpu.make_async_copy(loc_rows, out_h.at[rows, pl.ds(0, LOC_DIM)], wsem).wait()
        pltpu.make_async_copy(small_rows, out_h.at[rows, pl.ds(LOC_DIM, SMALL_DIM)], wsem).wait()

    def process(ci, slot):
        (src_v, week_v, hour_v, dur_v, loc_rows, small_rows,
         isem, gsem, wsem) = slot
        cps = []
        for j in range(SEGS_PER_CHUNK):
            d = pl.ds(j * SEG, SEG)
            cps.append(pltpu.async_copy(loc_t.at[src_v.at[j]], loc_rows.at[d], gsem))
        for j in range(SEGS_PER_CHUNK):
            def fill(g, carry):
                lanes = pl.ds(g * 16, 16)
                w16 = week_v[j, lanes] * 16
                h16 = hour_v[j, lanes] * 16
                d16 = dur_v[j, lanes] * 16
                rows16 = lax.iota(jnp.int32, 16) + (j * SEG + g * 16)
                for j2 in range(16):
                    col = jnp.full((16,), j2, jnp.int32)
                    plsc.store_scatter(small_rows, [rows16, col],
                                       plsc.load_gather(week_tv, [w16 + j2]))
                    plsc.store_scatter(small_rows, [rows16, col + 16],
                                       plsc.load_gather(hour_tv, [h16 + j2]))
                    plsc.store_scatter(small_rows, [rows16, col + 32],
                                       plsc.load_gather(dur_tv, [d16 + j2]))
                return carry
            lax.fori_loop(0, SEG // 16, fill, 0)
        for c in cps:
            c.wait()
        tok0 = (row0 + ci * SEGS_PER_CHUNK) * SEG
        rows = pl.ds(tok0, CHUNK)
        pltpu.async_copy(loc_rows, out_h.at[rows, pl.ds(0, LOC_DIM)], wsem)
        pltpu.async_copy(small_rows, out_h.at[rows, pl.ds(LOC_DIM, SMALL_DIM)], wsem)

    stage_idx(0, slot_a)

    def pair_body(k, carry):
        ca = 2 * k
        cb = 2 * k + 1
        pl.when(k > 0)(lambda: wait_writes(ca - 2, slot_a))
        wait_idx(slot_a)
        stage_idx(cb, slot_b)
        process(ca, slot_a)
        pl.when(k > 0)(lambda: wait_writes(cb - 2, slot_b))
        wait_idx(slot_b)
        pl.when(k < PAIRS_PER_W - 1)(lambda: stage_idx(cb + 1, slot_a))
        process(cb, slot_b)
        return carry

    lax.fori_loop(0, PAIRS_PER_W, pair_body, 0)
    wait_writes(CHUNKS_PER_W - 2, slot_a)
    wait_writes(CHUNKS_PER_W - 1, slot_b)


def kernel(src, week, hour, duration, loc_table, week_table, hour_table, duration_table):
    src2 = src.reshape(N // SEG, SEG).astype(jnp.int32)
    week2 = week.reshape(N // SEG, SEG).astype(jnp.int32)
    hour2 = hour.reshape(N // SEG, SEG).astype(jnp.int32)
    dur2 = duration.reshape(N // SEG, SEG).astype(jnp.int32)

    mesh = plsc.VectorSubcoreMesh(core_axis_name="c", subcore_axis_name="s",
                                  num_cores=NC, num_subcores=NS)
    idx_t = pltpu.VMEM((SEGS_PER_CHUNK, SEG), jnp.int32)
    run = pl.kernel(
        _sc_kernel,
        out_type=jax.ShapeDtypeStruct((N, 128), jnp.float32),
        mesh=mesh,
        compiler_params=pltpu.CompilerParams(use_tc_tiling_on_sc=False,
                                             needs_layout_passes=False),
        scratch_types=(
            [idx_t] * 4 + [pltpu.VMEM((CHUNK, LOC_DIM), jnp.float32),
                           pltpu.VMEM((CHUNK, SMALL_DIM), jnp.float32)]
        ) * 2 + [
            pltpu.VMEM((7 * 16,), jnp.float32),
            pltpu.VMEM((24 * 16,), jnp.float32),
            pltpu.VMEM((24 * 16,), jnp.float32),
        ] + [pltpu.SemaphoreType.DMA] * 6,
    )
    out = run(src2, week2, hour2, dur2, loc_table,
              week_table.reshape(-1), hour_table.reshape(-1),
              duration_table.reshape(-1))
    return out[:, :OUT_DIM].reshape(B, T, OUT_DIM)


# speedup vs baseline: 1.3008x; 1.3008x over previous
import jax
import jax.numpy as jnp
from jax import lax
from jax.experimental import pallas as pl
from jax.experimental.pallas import tpu as pltpu
from jax.experimental.pallas import tpu_sc as plsc

B, T = 4096, 200
N = B * T
LOC_DIM = 64
SMALL_DIM = 48
OUT_DIM = LOC_DIM + SMALL_DIM

NC, NS = 2, 16
NW = NC * NS
SEG = 128
SEGS_PER_CHUNK = 4
CHUNK = SEG * SEGS_PER_CHUNK
PER_W = N // NW
CHUNKS_PER_W = PER_W // CHUNK
PAIRS_PER_W = CHUNKS_PER_W // 2
ROWS_PER_W = PER_W // SEG


def _sc_kernel(src_h, week_h, hour_h, dur_h, loc_t, small_t, out_h,
               src_a, week_a, hour_a, dur_a, sidx_a, loc_rows_a, small_rows_a,
               src_b, week_b, hour_b, dur_b, sidx_b, loc_rows_b, small_rows_b,
               isem_a, isem_b, gsem_a, gsem_b, wsem_a, wsem_b):
    wid = lax.axis_index("s") * NC + lax.axis_index("c")
    row0 = wid * ROWS_PER_W

    slot_a = (src_a, week_a, hour_a, dur_a, sidx_a, loc_rows_a, small_rows_a,
              isem_a, gsem_a, wsem_a)
    slot_b = (src_b, week_b, hour_b, dur_b, sidx_b, loc_rows_b, small_rows_b,
              isem_b, gsem_b, wsem_b)

    def stage_idx(ci, slot):
        src_v, week_v, hour_v, dur_v = slot[0], slot[1], slot[2], slot[3]
        isem = slot[7]
        r = row0 + ci * SEGS_PER_CHUNK
        pltpu.async_copy(src_h.at[pl.ds(r, SEGS_PER_CHUNK)], src_v, isem)
        pltpu.async_copy(week_h.at[pl.ds(r, SEGS_PER_CHUNK)], week_v, isem)
        pltpu.async_copy(hour_h.at[pl.ds(r, SEGS_PER_CHUNK)], hour_v, isem)
        pltpu.async_copy(dur_h.at[pl.ds(r, SEGS_PER_CHUNK)], dur_v, isem)

    def wait_idx(slot):
        src_v, week_v, hour_v, dur_v = slot[0], slot[1], slot[2], slot[3]
        isem = slot[7]
        for v in (src_v, week_v, hour_v, dur_v):
            pltpu.make_async_copy(src_h.at[pl.ds(0, SEGS_PER_CHUNK)], v, isem).wait()

    def wait_writes(ci, slot):
        loc_rows, small_rows, wsem = slot[5], slot[6], slot[9]
        tok0 = (row0 + ci * SEGS_PER_CHUNK) * SEG
        rows = pl.ds(tok0, CHUNK)
        pltpu.make_async_copy(loc_rows, out_h.at[rows, pl.ds(0, LOC_DIM)], wsem).wait()
        pltpu.make_async_copy(small_rows, out_h.at[rows, pl.ds(LOC_DIM, SMALL_DIM)], wsem).wait()

    def process(ci, slot):
        (src_v, week_v, hour_v, dur_v, sidx_v, loc_rows, small_rows,
         isem, gsem, wsem) = slot
        for j in range(SEGS_PER_CHUNK):
            def fuse(k, carry):
                d = pl.ds(k * 16, 16)
                sidx_v[j, d] = (week_v[j, d] * 576 + hour_v[j, d] * 24
                                + dur_v[j, d])
                return carry
            lax.fori_loop(0, SEG // 16, fuse, 0)
        cps = []
        for j in range(SEGS_PER_CHUNK):
            d = pl.ds(j * SEG, SEG)
            cps.append(pltpu.async_copy(loc_t.at[src_v.at[j]], loc_rows.at[d], gsem))
            cps.append(pltpu.async_copy(small_t.at[sidx_v.at[j]], small_rows.at[d], gsem))
        for c in cps:
            c.wait()
        tok0 = (row0 + ci * SEGS_PER_CHUNK) * SEG
        rows = pl.ds(tok0, CHUNK)
        pltpu.async_copy(loc_rows, out_h.at[rows, pl.ds(0, LOC_DIM)], wsem)
        pltpu.async_copy(small_rows, out_h.at[rows, pl.ds(LOC_DIM, SMALL_DIM)], wsem)

    stage_idx(0, slot_a)

    def pair_body(k, carry):
        ca = 2 * k
        cb = 2 * k + 1
        pl.when(k > 0)(lambda: wait_writes(ca - 2, slot_a))
        wait_idx(slot_a)
        stage_idx(cb, slot_b)
        process(ca, slot_a)
        pl.when(k > 0)(lambda: wait_writes(cb - 2, slot_b))
        wait_idx(slot_b)
        pl.when(k < PAIRS_PER_W - 1)(lambda: stage_idx(cb + 1, slot_a))
        process(cb, slot_b)
        return carry

    lax.fori_loop(0, PAIRS_PER_W, pair_body, 0)
    wait_writes(CHUNKS_PER_W - 2, slot_a)
    wait_writes(CHUNKS_PER_W - 1, slot_b)


def kernel(src, week, hour, duration, loc_table, week_table, hour_table, duration_table):
    src2 = src.reshape(N // SEG, SEG).astype(jnp.int32)
    week2 = week.reshape(N // SEG, SEG).astype(jnp.int32)
    hour2 = hour.reshape(N // SEG, SEG).astype(jnp.int32)
    dur2 = duration.reshape(N // SEG, SEG).astype(jnp.int32)

    fused = jnp.concatenate([
        jnp.broadcast_to(week_table[:, None, None, :], (7, 24, 24, 16)),
        jnp.broadcast_to(hour_table[None, :, None, :], (7, 24, 24, 16)),
        jnp.broadcast_to(duration_table[None, None, :, :], (7, 24, 24, 16)),
    ], axis=-1).reshape(7 * 24 * 24, SMALL_DIM)

    mesh = plsc.VectorSubcoreMesh(core_axis_name="c", subcore_axis_name="s",
                                  num_cores=NC, num_subcores=NS)
    idx_t = pltpu.VMEM((SEGS_PER_CHUNK, SEG), jnp.int32)
    run = pl.kernel(
        _sc_kernel,
        out_type=jax.ShapeDtypeStruct((N, 128), jnp.float32),
        mesh=mesh,
        compiler_params=pltpu.CompilerParams(use_tc_tiling_on_sc=False),
        scratch_types=(
            [idx_t] * 5 + [pltpu.VMEM((CHUNK, LOC_DIM), jnp.float32),
                           pltpu.VMEM((CHUNK, SMALL_DIM), jnp.float32)]
        ) * 2 + [pltpu.SemaphoreType.DMA] * 6,
    )
    out = run(src2, week2, hour2, dur2, loc_table, fused)
    return out[:, :OUT_DIM].reshape(B, T, OUT_DIM)
